# baseline (speedup 1.0000x reference)
import functools

import jax
import jax.numpy as jnp
from jax import lax
from jax.experimental import pallas as pl
from jax.experimental.pallas import tpu as pltpu
from jax.experimental.pallas import tpu_sc as plsc

NC, NS, LANES = 2, 16, 16
NW = NC * NS
ROWS, COLS = 4 * 4096, 2048
RPW = ROWS // NW
T = 16
NT = RPW // T


def _permute_body(param_hbm, perm_hbm, out_hbm, perm_v,
                  in0, in1, out0, si0, si1, so0):
    wid = lax.axis_index("s") * NC + lax.axis_index("c")
    base = wid * RPW
    pltpu.sync_copy(perm_hbm, perm_v)

    def start_in(t, buf, sem):
        pltpu.async_copy(param_hbm.at[pl.ds(base + t * T, T)], buf, sem)

    def wait_in(buf, sem):
        pltpu.make_async_copy(param_hbm.at[pl.ds(0, T)], buf, sem).wait()

    def start_out(t, sem):
        pltpu.async_copy(out0, out_hbm.at[pl.ds(base + t * T, T)], sem)

    def wait_out(sem):
        pltpu.make_async_copy(out0, out_hbm.at[pl.ds(0, T)], sem).wait()

    def compute(in_buf, out_buf):
        @plsc.parallel_loop(0, COLS // LANES, unroll=4)
        def _(j):
            j16 = j * LANES
            idx = perm_v[pl.ds(j16, LANES)]
            for r in range(T):
                row = jnp.full((LANES,), r, jnp.int32)
                out_buf[r, pl.ds(j16, LANES)] = plsc.load_gather(
                    in_buf, [row, idx]
                )

    start_in(0, in0, si0)
    start_in(1, in1, si1)

    def g_body(g, carry):
        t0 = 2 * g

        wait_in(in0, si0)
        pl.when(g > 0)(lambda: wait_out(so0))
        compute(in0, out0)
        start_out(t0, so0)
        pl.when(t0 + 2 < NT)(lambda: start_in(t0 + 2, in0, si0))

        wait_in(in1, si1)
        wait_out(so0)
        compute(in1, out0)
        start_out(t0 + 1, so0)
        pl.when(t0 + 3 < NT)(lambda: start_in(t0 + 3, in1, si1))
        return carry

    lax.fori_loop(0, NT // 2, g_body, 0)
    wait_out(so0)


@jax.jit
def kernel(param, permutation):
    p2 = param.reshape(ROWS, COLS)
    perm = permutation.astype(jnp.int32)
    run = pl.kernel(
        _permute_body,
        out_type=jax.ShapeDtypeStruct((ROWS, COLS), jnp.float32),
        mesh=plsc.VectorSubcoreMesh(
            core_axis_name="c", subcore_axis_name="s",
            num_cores=NC, num_subcores=NS,
        ),
        scratch_types=[
            pltpu.VMEM((COLS,), jnp.int32),
            pltpu.VMEM((T, COLS), jnp.float32),
            pltpu.VMEM((T, COLS), jnp.float32),
            pltpu.VMEM((T, COLS), jnp.float32),
            pltpu.SemaphoreType.DMA,
            pltpu.SemaphoreType.DMA,
            pltpu.SemaphoreType.DMA,
        ],
        compiler_params=pltpu.CompilerParams(needs_layout_passes=False),
    )
    out = run(p2, perm)
    return out.reshape(param.shape)


# speedup vs baseline: 1.0654x; 1.0654x over previous
import functools

import jax
import jax.numpy as jnp
from jax import lax
from jax.experimental import pallas as pl
from jax.experimental.pallas import tpu as pltpu
from jax.experimental.pallas import tpu_sc as plsc

NC, NS, LANES = 2, 16, 16
NW = NC * NS
ROWS, COLS = 4 * 4096, 2048
RPW = ROWS // NW
T = 8
NT = RPW // T
NB = 3
NG = NT // NB
assert NT % NB == 1


def _permute_body(param_hbm, perm_hbm, out_hbm, perm_v,
                  ins, outs, sis, sos):
    wid = lax.axis_index("s") * NC + lax.axis_index("c")
    base = wid * RPW
    pltpu.sync_copy(perm_hbm, perm_v)

    def start_in(t, b):
        pltpu.async_copy(param_hbm.at[pl.ds(base + t * T, T)], ins[b], sis[b])

    def wait_in(b):
        pltpu.make_async_copy(param_hbm.at[pl.ds(0, T)], ins[b], sis[b]).wait()

    def start_out(t, b):
        pltpu.async_copy(outs[b], out_hbm.at[pl.ds(base + t * T, T)], sos[b])

    def wait_out(b):
        pltpu.make_async_copy(outs[b], out_hbm.at[pl.ds(0, T)], sos[b]).wait()

    def compute(in_buf, out_buf):
        @plsc.parallel_loop(0, COLS // LANES, unroll=4)
        def _(j):
            j16 = j * LANES
            idx = perm_v[pl.ds(j16, LANES)]
            for r in range(T):
                row = jnp.full((LANES,), r, jnp.int32)
                out_buf[r, pl.ds(j16, LANES)] = plsc.load_gather(
                    in_buf, [row, idx]
                )

    for b in range(NB):
        start_in(b, b)

    def g_body(g, carry):
        for b in range(NB):
            t = NB * g + b
            wait_in(b)
            pl.when(g > 0)(lambda: wait_out(b))
            compute(ins[b], outs[b])
            start_out(t, b)
            pl.when(t + NB < NT)(lambda: start_in(t + NB, b))
        return carry

    lax.fori_loop(0, NG, g_body, 0)

    wait_in(0)
    wait_out(0)
    compute(ins[0], outs[0])
    start_out(NT - 1, 0)

    for b in range(NB):
        wait_out(b)


@jax.jit
def kernel(param, permutation):
    p2 = param.reshape(ROWS, COLS)
    perm = permutation.astype(jnp.int32)

    def body(param_hbm, perm_hbm, out_hbm, perm_v,
             in0, in1, in2, out0, out1, out2,
             si0, si1, si2, so0, so1, so2):
        _permute_body(param_hbm, perm_hbm, out_hbm, perm_v,
                      [in0, in1, in2], [out0, out1, out2],
                      [si0, si1, si2], [so0, so1, so2])

    run = pl.kernel(
        body,
        out_type=jax.ShapeDtypeStruct((ROWS, COLS), jnp.float32),
        mesh=plsc.VectorSubcoreMesh(
            core_axis_name="c", subcore_axis_name="s",
            num_cores=NC, num_subcores=NS,
        ),
        scratch_types=[
            pltpu.VMEM((COLS,), jnp.int32),
            pltpu.VMEM((T, COLS), jnp.float32),
            pltpu.VMEM((T, COLS), jnp.float32),
            pltpu.VMEM((T, COLS), jnp.float32),
            pltpu.VMEM((T, COLS), jnp.float32),
            pltpu.VMEM((T, COLS), jnp.float32),
            pltpu.VMEM((T, COLS), jnp.float32),
            pltpu.SemaphoreType.DMA,
            pltpu.SemaphoreType.DMA,
            pltpu.SemaphoreType.DMA,
            pltpu.SemaphoreType.DMA,
            pltpu.SemaphoreType.DMA,
            pltpu.SemaphoreType.DMA,
        ],
        compiler_params=pltpu.CompilerParams(
            needs_layout_passes=False,
            disable_bounds_checks=True,
            skip_device_barrier=True,
        ),
    )
    out = run(p2, perm)
    return out.reshape(param.shape)
